# baseline (speedup 1.0000x reference)
import jax
import jax.numpy as jnp
from jax import lax
from jax.experimental import pallas as pl
from jax.experimental.pallas import tpu as pltpu
from jax.experimental.pallas import tpu_sc as plsc

B, T = 16, 4096
P = 8
D = 16
NC, NS = 2, 16
NW = NC * NS
NPATCH = (B * T) // P
PPW = NPATCH // NW
TPW = PPW * P
HALF = PPW // 2
L = 16
NSTAGE = 8
ROWS_PER_STAGE = 256 // NSTAGE


def _sc_body(table_hbm, x_hbm, out_hbm, table_sh, x_v, idx_v, acc_v, tbl_v,
             sem_h0, sem_h1, sem_x, sem_out):
    sid = lax.axis_index("s")
    wid = sid * NC + lax.axis_index("c")
    b = wid // 2
    half_id = wid % 2

    xc = pltpu.async_copy(x_hbm.at[b, pl.ds(half_id * TPW, TPW)], x_v, sem_x)

    scale = jnp.full((D,), 1.0 / P, dtype=jnp.float32)

    @pl.when(sid < NSTAGE)
    def _():
        r0 = sid * ROWS_PER_STAGE
        pltpu.sync_copy(table_hbm.at[pl.ds(r0, ROWS_PER_STAGE)], tbl_v)
        for i in range(ROWS_PER_STAGE):
            tbl_v[i, :] = tbl_v[i, :] * scale
        pltpu.sync_copy(tbl_v, table_sh.at[pl.ds(r0, ROWS_PER_STAGE)])

    zero = jnp.zeros((D,), jnp.float32)

    def zero_body(i, _):
        acc_v[i, :] = zero
        return 0

    lax.fori_loop(0, PPW, zero_body, 0, unroll=8)

    xc.wait()

    lane8 = lax.iota(jnp.int32, L) * P
    for h in range(2):
        for t in range(P):
            base = h * HALF * P + t
            for j0 in range(0, HALF, L):
                idx_v[h, t, pl.ds(j0, L)] = plsc.load_gather(
                    x_v, [lane8 + (base + j0 * P)])

    plsc.subcore_barrier()

    sems = (sem_h0, sem_h1)
    adds = [
        pltpu.async_copy(table_sh.at[idx_v.at[h, t]],
                         acc_v.at[pl.ds(h * HALF, HALF)], sems[h], add=True)
        for h in range(2)
        for t in range(P)
    ]
    outs = []
    for h in range(2):
        for c in adds[h * P:(h + 1) * P]:
            c.wait()
        outs.append(pltpu.async_copy(
            acc_v.at[pl.ds(h * HALF, HALF)],
            out_hbm.at[b, pl.ds(half_id * PPW + h * HALF, HALF), :],
            sem_out))
    for c in outs:
        c.wait()


_mesh = plsc.VectorSubcoreMesh(
    core_axis_name="c", subcore_axis_name="s", num_cores=NC, num_subcores=NS
)

_patch_pool = pl.kernel(
    _sc_body,
    out_type=jax.ShapeDtypeStruct((B, T // P, D), jnp.float32),
    mesh=_mesh,
    scratch_types=[
        pltpu.VMEM_SHARED((256, D), jnp.float32),
        pltpu.VMEM((TPW,), jnp.int32),
        pltpu.VMEM((2, P, HALF), jnp.int32),
        pltpu.VMEM((PPW, D), jnp.float32),
        pltpu.VMEM((ROWS_PER_STAGE, D), jnp.float32),
        pltpu.SemaphoreType.DMA,
        pltpu.SemaphoreType.DMA,
        pltpu.SemaphoreType.DMA,
        pltpu.SemaphoreType.DMA,
    ],
    compiler_params=pltpu.CompilerParams(
        use_tc_tiling_on_sc=False, needs_layout_passes=False),
)


def kernel(x_u8, byte_embed):
    return _patch_pool(byte_embed, x_u8)


# speedup vs baseline: 1.0104x; 1.0104x over previous
import jax
import jax.numpy as jnp
from jax import lax
from jax.experimental import pallas as pl
from jax.experimental.pallas import tpu as pltpu
from jax.experimental.pallas import tpu_sc as plsc

B, T = 16, 4096
P = 8
D = 16
NC, NS = 2, 16
NW = NC * NS
NPATCH = (B * T) // P
PPW = NPATCH // NW
TPW = PPW * P
HALF = PPW // 2
L = 16
NSTAGE = 16
ROWS_PER_STAGE = 256 // NSTAGE


def _sc_body(table_hbm, x_hbm, out_hbm, table_sh, x_v, idx_v, acc_v, tbl_v,
             sem, sem_x, sem_t):
    sid = lax.axis_index("s")
    wid = sid * NC + lax.axis_index("c")
    b = wid // 2
    half_id = wid % 2

    xc = pltpu.async_copy(x_hbm.at[b, pl.ds(half_id * TPW, TPW)], x_v, sem_x)

    r0 = sid * ROWS_PER_STAGE
    tc = pltpu.async_copy(
        table_hbm.at[pl.ds(r0, ROWS_PER_STAGE)], tbl_v, sem_t)

    zero = jnp.zeros((D,), jnp.float32)

    def zero_body(i, _):
        acc_v[i, :] = zero
        return 0

    lax.fori_loop(0, PPW, zero_body, 0, unroll=8)

    scale = jnp.full((D,), 1.0 / P, dtype=jnp.float32)
    tc.wait()
    for i in range(ROWS_PER_STAGE):
        tbl_v[i, :] = tbl_v[i, :] * scale
    pltpu.sync_copy(tbl_v, table_sh.at[pl.ds(r0, ROWS_PER_STAGE)])

    xc.wait()

    lane8 = lax.iota(jnp.int32, L) * P
    for t in range(P):
        for j0 in range(0, PPW, L):
            idx_v[t, pl.ds(j0, L)] = plsc.load_gather(
                x_v, [lane8 + (j0 * P + t)])

    plsc.subcore_barrier()

    adds = [
        pltpu.async_copy(table_sh.at[idx_v.at[t]], acc_v, sem, add=True)
        for t in range(P)
    ]
    for c in adds:
        c.wait()

    pltpu.sync_copy(acc_v, out_hbm.at[b, pl.ds(half_id * PPW, PPW), :])


_mesh = plsc.VectorSubcoreMesh(
    core_axis_name="c", subcore_axis_name="s", num_cores=NC, num_subcores=NS
)

_patch_pool = pl.kernel(
    _sc_body,
    out_type=jax.ShapeDtypeStruct((B, T // P, D), jnp.float32),
    mesh=_mesh,
    scratch_types=[
        pltpu.VMEM_SHARED((256, D), jnp.float32),
        pltpu.VMEM((TPW,), jnp.int32),
        pltpu.VMEM((P, PPW), jnp.int32),
        pltpu.VMEM((PPW, D), jnp.float32),
        pltpu.VMEM((ROWS_PER_STAGE, D), jnp.float32),
        pltpu.SemaphoreType.DMA,
        pltpu.SemaphoreType.DMA,
        pltpu.SemaphoreType.DMA,
    ],
    compiler_params=pltpu.CompilerParams(
        use_tc_tiling_on_sc=False, needs_layout_passes=False),
)


def kernel(x_u8, byte_embed):
    return _patch_pool(byte_embed, x_u8)
